# baseline (speedup 1.0000x reference)
import functools

import jax
import jax.numpy as jnp
from jax import lax
from jax.experimental import pallas as pl
from jax.experimental.pallas import tpu as pltpu
from jax.experimental.pallas import tpu_sc as plsc

NT = 1000000
D = 64
B = 16384
NSLAB = NT // 4
NW = 32
BPW = B // NW

_sc_mesh = plsc.VectorSubcoreMesh(core_axis_name="c", subcore_axis_name="s")


@functools.partial(
    pl.kernel,
    out_type=(
        jax.ShapeDtypeStruct((B, 128), jnp.int32),
        jax.ShapeDtypeStruct((B, 128), jnp.int32),
    ),
    mesh=_sc_mesh,
    scratch_types=[
        pltpu.VMEM((BPW,), jnp.int32),
        pltpu.VMEM((BPW,), jnp.int32),
        pltpu.VMEM((BPW, 128), jnp.int32),
        pltpu.SemaphoreType.DMA,
        pltpu.SemaphoreType.DMA,
    ],
)
def _sc_gather(uid_hbm, iid_hbm, ut_hbm, it_hbm, su_hbm, si_hbm,
               idx_v, sidx_v, rows_v, sem_g, sem_s):
    wid = lax.axis_index("s") * 2 + lax.axis_index("c")
    base = wid * BPW

    def run_table(ids_hbm, tab_hbm, out_hbm):
        pltpu.async_copy(ids_hbm.at[pl.ds(base, BPW)], idx_v, sem_s).wait()

        @pl.loop(0, BPW, step=16)
        def _(i):
            v = idx_v[pl.ds(i, 16)]
            sidx_v[pl.ds(i, 16)] = lax.shift_right_logical(v, 2)

        pltpu.async_copy(tab_hbm.at[sidx_v], rows_v, sem_g).wait()
        pltpu.sync_copy(rows_v, out_hbm.at[pl.ds(base, BPW)])

    run_table(uid_hbm, ut_hbm, su_hbm)
    run_table(iid_hbm, it_hbm, si_hbm)


BLK = 2048


def _mlp_body(su_ref, si_ref, hu_ref, hi_ref, w1ue_ref, w1uo_ref, w1ie_ref,
              w1io_ref, b1_ref, w2_ref, b2_ref, w3_ref, b3_ref, out_ref):
    def contrib(s_ref, h_ref, we_ref, wo_ref):
        x = s_ref[...]
        le = lax.bitcast_convert_type(lax.shift_left(x, 16), jnp.float32)
        lo = lax.bitcast_convert_type(lax.bitwise_and(x, jnp.int32(-65536)), jnp.float32)
        h = h_ref[...]
        acc = None
        for k in range(4):
            ek = le[:, 32 * k:32 * k + 32]
            ok = lo[:, 32 * k:32 * k + 32]
            ak = (jnp.dot(ek, we_ref[...], preferred_element_type=jnp.float32)
                  + jnp.dot(ok, wo_ref[...], preferred_element_type=jnp.float32))
            sk = (h == float(k)).astype(jnp.float32)
            acc = ak * sk if acc is None else acc + ak * sk
        return acc

    x1 = (contrib(su_ref, hu_ref, w1ue_ref, w1uo_ref)
          + contrib(si_ref, hi_ref, w1ie_ref, w1io_ref) + b1_ref[...])
    h1 = jnp.maximum(x1, 0.0)
    h2 = jnp.maximum(
        jnp.dot(h1, w2_ref[...], preferred_element_type=jnp.float32) + b2_ref[...],
        0.0)
    z = jnp.sum(h2 * w3_ref[...], axis=1, keepdims=True) + b3_ref[0, 0]
    out_ref[...] = 1.0 / (1.0 + jnp.exp(-z))


def _mlp(su, si, hu, hi, w1ue, w1uo, w1ie, w1io, b1r, w2t, b2r, w3r, b3r):
    grid = (B // BLK,)
    full = lambda shape: pl.BlockSpec(shape, lambda i: (0,) * len(shape))
    return pl.pallas_call(
        _mlp_body,
        grid=grid,
        in_specs=[
            pl.BlockSpec((BLK, 128), lambda i: (i, 0)),
            pl.BlockSpec((BLK, 128), lambda i: (i, 0)),
            pl.BlockSpec((BLK, 1), lambda i: (i, 0)),
            pl.BlockSpec((BLK, 1), lambda i: (i, 0)),
            full((32, 128)),
            full((32, 128)),
            full((32, 128)),
            full((32, 128)),
            full((1, 128)),
            full((128, D)),
            full((1, D)),
            full((1, D)),
            full((1, 1)),
        ],
        out_specs=pl.BlockSpec((BLK, 1), lambda i: (i, 0)),
        out_shape=jax.ShapeDtypeStruct((B, 1), jnp.float32),
    )(su, si, hu, hi, w1ue, w1uo, w1ie, w1io, b1r, w2t, b2r, w3r, b3r)


def _pack_table(tab):
    b = jnp.reshape(tab.astype(jnp.bfloat16), (NSLAB, 128, 2))
    return lax.bitcast_convert_type(b, jnp.int32)


def kernel(user_ids, item_ids, user_table, item_table, W1, b1, W2, b2, W3, b3):
    uid = user_ids.astype(jnp.int32)
    iid = item_ids.astype(jnp.int32)
    u32 = _pack_table(user_table)
    i32 = _pack_table(item_table)
    su, si = _sc_gather(uid, iid, u32, i32)

    w1u = W1[:, :D].T
    w1i = W1[:, D:].T
    w1ue, w1uo = w1u[0::2, :], w1u[1::2, :]
    w1ie, w1io = w1i[0::2, :], w1i[1::2, :]
    hu = (uid & 3).astype(jnp.float32).reshape(B, 1)
    hi = (iid & 3).astype(jnp.float32).reshape(B, 1)

    out = _mlp(su, si, hu, hi, w1ue, w1uo, w1ie, w1io,
               b1.reshape(1, 128), W2.T, b2.reshape(1, D), W3,
               b3.reshape(1, 1))
    return out[:, 0]


# speedup vs baseline: 12.2767x; 12.2767x over previous
import functools

import jax
import jax.numpy as jnp
from jax import lax
from jax.experimental import pallas as pl
from jax.experimental.pallas import tpu as pltpu
from jax.experimental.pallas import tpu_sc as plsc

NT = 1000000
D = 64
B = 16384
NSLAB = NT // 4
NW = 32
BPW = B // NW

_sc_mesh = plsc.VectorSubcoreMesh(core_axis_name="c", subcore_axis_name="s")


@functools.partial(
    pl.kernel,
    out_type=(
        jax.ShapeDtypeStruct((B, 128), jnp.int32),
        jax.ShapeDtypeStruct((B, 128), jnp.int32),
    ),
    mesh=_sc_mesh,
    scratch_types=[
        pltpu.VMEM((BPW,), jnp.int32),
        pltpu.VMEM((BPW,), jnp.int32),
        pltpu.VMEM((BPW, 128), jnp.int32),
        pltpu.SemaphoreType.DMA,
        pltpu.SemaphoreType.DMA,
    ],
)
def _sc_gather(uid_hbm, iid_hbm, ut_hbm, it_hbm, su_hbm, si_hbm,
               idx_v, sidx_v, rows_v, sem_g, sem_s):
    wid = lax.axis_index("s") * 2 + lax.axis_index("c")
    base = wid * BPW

    def run_table(ids_hbm, tab_hbm, out_hbm):
        pltpu.async_copy(ids_hbm.at[pl.ds(base, BPW)], idx_v, sem_s).wait()

        @pl.loop(0, BPW, step=16)
        def _(i):
            v = idx_v[pl.ds(i, 16)]
            sidx_v[pl.ds(i, 16)] = lax.shift_right_logical(v, 2)

        pltpu.async_copy(tab_hbm.at[sidx_v], rows_v, sem_g).wait()
        pltpu.sync_copy(rows_v, out_hbm.at[pl.ds(base, BPW)])

    run_table(uid_hbm, ut_hbm, su_hbm)
    run_table(iid_hbm, it_hbm, si_hbm)


BLK = 2048


def _mlp_body(su_ref, si_ref, hu_ref, hi_ref, w1ue_ref, w1uo_ref, w1ie_ref,
              w1io_ref, b1_ref, w2_ref, b2_ref, w3_ref, b3_ref, out_ref):
    def contrib(s_ref, h_ref, we_ref, wo_ref):
        x = s_ref[...]
        le = lax.bitcast_convert_type(lax.shift_left(x, 16), jnp.float32)
        lo = lax.bitcast_convert_type(lax.bitwise_and(x, jnp.int32(-65536)), jnp.float32)
        h = h_ref[...]
        acc = None
        for k in range(4):
            ek = le[:, 32 * k:32 * k + 32]
            ok = lo[:, 32 * k:32 * k + 32]
            ak = (jnp.dot(ek, we_ref[...], preferred_element_type=jnp.float32)
                  + jnp.dot(ok, wo_ref[...], preferred_element_type=jnp.float32))
            sk = (h == float(k)).astype(jnp.float32)
            acc = ak * sk if acc is None else acc + ak * sk
        return acc

    x1 = (contrib(su_ref, hu_ref, w1ue_ref, w1uo_ref)
          + contrib(si_ref, hi_ref, w1ie_ref, w1io_ref) + b1_ref[...])
    h1 = jnp.maximum(x1, 0.0)
    h2 = jnp.maximum(
        jnp.dot(h1, w2_ref[...], preferred_element_type=jnp.float32) + b2_ref[...],
        0.0)
    z = jnp.sum(h2 * w3_ref[...], axis=1, keepdims=True) + b3_ref[0, 0]
    out_ref[...] = 1.0 / (1.0 + jnp.exp(-z))


def _mlp(su, si, hu, hi, w1ue, w1uo, w1ie, w1io, b1r, w2t, b2r, w3r, b3r):
    grid = (B // BLK,)
    full = lambda shape: pl.BlockSpec(shape, lambda i: (0,) * len(shape))
    return pl.pallas_call(
        _mlp_body,
        grid=grid,
        in_specs=[
            pl.BlockSpec((BLK, 128), lambda i: (i, 0)),
            pl.BlockSpec((BLK, 128), lambda i: (i, 0)),
            pl.BlockSpec((BLK, 1), lambda i: (i, 0)),
            pl.BlockSpec((BLK, 1), lambda i: (i, 0)),
            full((32, 128)),
            full((32, 128)),
            full((32, 128)),
            full((32, 128)),
            full((1, 128)),
            full((128, D)),
            full((1, D)),
            full((1, D)),
            full((1, 1)),
        ],
        out_specs=pl.BlockSpec((BLK, 1), lambda i: (i, 0)),
        out_shape=jax.ShapeDtypeStruct((B, 1), jnp.float32),
    )(su, si, hu, hi, w1ue, w1uo, w1ie, w1io, b1r, w2t, b2r, w3r, b3r)


def _bf16_bits(x):
    b = lax.bitcast_convert_type(x, jnp.int32)
    rounded = b + jnp.int32(0x7FFF) + (lax.shift_right_logical(b, 16) & 1)
    return lax.shift_right_logical(rounded, 16)


def _pack_table(tab):
    e = _bf16_bits(tab[:, 0::2])
    o = _bf16_bits(tab[:, 1::2])
    p = e | lax.shift_left(o, 16)
    return jnp.reshape(p, (NSLAB, 128))


def kernel(user_ids, item_ids, user_table, item_table, W1, b1, W2, b2, W3, b3):
    uid = user_ids.astype(jnp.int32)
    iid = item_ids.astype(jnp.int32)
    u32 = _pack_table(user_table)
    i32 = _pack_table(item_table)
    su, si = _sc_gather(uid, iid, u32, i32)

    w1u = W1[:, :D].T
    w1i = W1[:, D:].T
    w1ue, w1uo = w1u[0::2, :], w1u[1::2, :]
    w1ie, w1io = w1i[0::2, :], w1i[1::2, :]
    hu = (uid & 3).astype(jnp.float32).reshape(B, 1)
    hi = (iid & 3).astype(jnp.float32).reshape(B, 1)

    out = _mlp(su, si, hu, hi, w1ue, w1uo, w1ie, w1io,
               b1.reshape(1, 128), W2.T, b2.reshape(1, D), W3,
               b3.reshape(1, 1))
    return out[:, 0]


# speedup vs baseline: 36.4410x; 2.9683x over previous
import functools

import jax
import jax.numpy as jnp
from jax import lax
from jax.experimental import pallas as pl
from jax.experimental.pallas import tpu as pltpu
from jax.experimental.pallas import tpu_sc as plsc

NT = 1000000
D = 64
B = 16384
NSLAB = NT // 4
NW = 32
BPW = B // NW
HB = BPW // 2

_sc_mesh = plsc.VectorSubcoreMesh(core_axis_name="c", subcore_axis_name="s")


@functools.partial(
    pl.kernel,
    out_type=(
        jax.ShapeDtypeStruct((B, 256), jnp.float32),
        jax.ShapeDtypeStruct((B, 256), jnp.float32),
    ),
    mesh=_sc_mesh,
    scratch_types=[
        pltpu.VMEM((BPW,), jnp.int32),
        pltpu.VMEM((BPW,), jnp.int32),
        pltpu.VMEM((HB, 256), jnp.float32),
        pltpu.SemaphoreType.DMA,
        pltpu.SemaphoreType.DMA,
    ],
)
def _sc_gather(uid_hbm, iid_hbm, ut_hbm, it_hbm, su_hbm, si_hbm,
               idx_v, sidx_v, rows_v, sem_g, sem_s):
    wid = lax.axis_index("s") * 2 + lax.axis_index("c")
    base = wid * BPW

    def run_table(ids_hbm, tab_hbm, out_hbm):
        pltpu.async_copy(ids_hbm.at[pl.ds(base, BPW)], idx_v, sem_s).wait()

        @pl.loop(0, BPW, step=16)
        def _(i):
            v = idx_v[pl.ds(i, 16)]
            sidx_v[pl.ds(i, 16)] = lax.shift_right_logical(v, 2)

        for half in range(2):
            pltpu.async_copy(
                tab_hbm.at[sidx_v.at[pl.ds(half * HB, HB)]], rows_v,
                sem_g).wait()
            pltpu.sync_copy(rows_v, out_hbm.at[pl.ds(base + half * HB, HB)])

    run_table(uid_hbm, ut_hbm, su_hbm)
    run_table(iid_hbm, it_hbm, si_hbm)


BLK = 2048


def _mlp_body(su_ref, si_ref, hu_ref, hi_ref, w1u_ref, w1i_ref,
              b1_ref, w2_ref, b2_ref, w3_ref, b3_ref, out_ref):
    def contrib(s_ref, h_ref, w_ref):
        q = s_ref[...]
        h = h_ref[...]
        acc = None
        for k in range(4):
            ak = jnp.dot(q[:, 64 * k:64 * k + 64], w_ref[...],
                         preferred_element_type=jnp.float32)
            sk = (h == float(k)).astype(jnp.float32)
            acc = ak * sk if acc is None else acc + ak * sk
        return acc

    x1 = (contrib(su_ref, hu_ref, w1u_ref)
          + contrib(si_ref, hi_ref, w1i_ref) + b1_ref[...])
    h1 = jnp.maximum(x1, 0.0)
    h2 = jnp.maximum(
        jnp.dot(h1, w2_ref[...], preferred_element_type=jnp.float32) + b2_ref[...],
        0.0)
    z = jnp.sum(h2 * w3_ref[...], axis=1, keepdims=True) + b3_ref[0, 0]
    out_ref[...] = 1.0 / (1.0 + jnp.exp(-z))


def _mlp(su, si, hu, hi, w1u, w1i, b1r, w2t, b2r, w3r, b3r):
    grid = (B // BLK,)
    full = lambda shape: pl.BlockSpec(shape, lambda i: (0,) * len(shape))
    return pl.pallas_call(
        _mlp_body,
        grid=grid,
        in_specs=[
            pl.BlockSpec((BLK, 256), lambda i: (i, 0)),
            pl.BlockSpec((BLK, 256), lambda i: (i, 0)),
            pl.BlockSpec((BLK, 1), lambda i: (i, 0)),
            pl.BlockSpec((BLK, 1), lambda i: (i, 0)),
            full((D, 128)),
            full((D, 128)),
            full((1, 128)),
            full((128, D)),
            full((1, D)),
            full((1, D)),
            full((1, 1)),
        ],
        out_specs=pl.BlockSpec((BLK, 1), lambda i: (i, 0)),
        out_shape=jax.ShapeDtypeStruct((B, 1), jnp.float32),
    )(su, si, hu, hi, w1u, w1i, b1r, w2t, b2r, w3r, b3r)


def kernel(user_ids, item_ids, user_table, item_table, W1, b1, W2, b2, W3, b3):
    uid = user_ids.astype(jnp.int32)
    iid = item_ids.astype(jnp.int32)
    u4 = jnp.reshape(user_table, (NSLAB, 256))
    i4 = jnp.reshape(item_table, (NSLAB, 256))
    su, si = _sc_gather(uid, iid, u4, i4)

    w1u = W1[:, :D].T
    w1i = W1[:, D:].T
    hu = (uid & 3).astype(jnp.float32).reshape(B, 1)
    hi = (iid & 3).astype(jnp.float32).reshape(B, 1)

    out = _mlp(su, si, hu, hi, w1u, w1i,
               b1.reshape(1, 128), W2.T, b2.reshape(1, D), W3,
               b3.reshape(1, 1))
    return out[:, 0]


# speedup vs baseline: 36.6485x; 1.0057x over previous
import functools

import jax
import jax.numpy as jnp
from jax import lax
from jax.experimental import pallas as pl
from jax.experimental.pallas import tpu as pltpu
from jax.experimental.pallas import tpu_sc as plsc

NT = 1000000
D = 64
B = 16384
NSLAB = NT // 4
NW = 32
BPW = B // NW
HB = BPW // 2

_sc_mesh = plsc.VectorSubcoreMesh(core_axis_name="c", subcore_axis_name="s")


@functools.partial(
    pl.kernel,
    out_type=(
        jax.ShapeDtypeStruct((B, 256), jnp.float32),
        jax.ShapeDtypeStruct((B, 256), jnp.float32),
    ),
    mesh=_sc_mesh,
    scratch_types=[
        pltpu.VMEM((BPW,), jnp.int32),
        pltpu.VMEM((BPW,), jnp.int32),
        pltpu.VMEM((HB, 256), jnp.float32),
        pltpu.SemaphoreType.DMA,
        pltpu.SemaphoreType.DMA,
    ],
)
def _sc_gather(uid_hbm, iid_hbm, ut_hbm, it_hbm, su_hbm, si_hbm,
               idx_v, sidx_v, rows_v, sem_g, sem_s):
    wid = lax.axis_index("s") * 2 + lax.axis_index("c")
    base = wid * BPW

    def run_table(ids_hbm, tab_hbm, out_hbm):
        pltpu.async_copy(ids_hbm.at[pl.ds(base, BPW)], idx_v, sem_s).wait()

        @pl.loop(0, BPW, step=16)
        def _(i):
            v = idx_v[pl.ds(i, 16)]
            sidx_v[pl.ds(i, 16)] = lax.shift_right_logical(v, 2)

        for half in range(2):
            pltpu.async_copy(
                tab_hbm.at[sidx_v.at[pl.ds(half * HB, HB)]], rows_v,
                sem_g).wait()
            pltpu.sync_copy(rows_v, out_hbm.at[pl.ds(base + half * HB, HB)])

    run_table(uid_hbm, ut_hbm, su_hbm)
    run_table(iid_hbm, it_hbm, si_hbm)


BLK = 2048


def _mlp_body(su_ref, si_ref, hu_ref, hi_ref, w1u_ref, w1i_ref,
              b1_ref, w2_ref, b2_ref, w3_ref, b3_ref, out_ref):
    def contrib(s_ref, h_ref, w_ref):
        q = s_ref[...]
        h = h_ref[...]
        acc = None
        for k in range(4):
            ak = jnp.dot(q[:, 64 * k:64 * k + 64], w_ref[...],
                         preferred_element_type=jnp.float32)
            sk = (h == float(k)).astype(jnp.float32)
            acc = ak * sk if acc is None else acc + ak * sk
        return acc

    x1 = (contrib(su_ref, hu_ref, w1u_ref)
          + contrib(si_ref, hi_ref, w1i_ref) + b1_ref[...])
    h1 = jnp.maximum(x1, 0.0)
    h2 = jnp.maximum(
        jnp.dot(h1, w2_ref[...], preferred_element_type=jnp.float32) + b2_ref[...],
        0.0)
    z = jnp.sum(h2 * w3_ref[...], axis=1, keepdims=True) + b3_ref[0, 0]
    out_ref[...] = 1.0 / (1.0 + jnp.exp(-z))


def _mlp(su, si, hu, hi, w1u, w1i, b1r, w2t, b2r, w3r, b3r):
    grid = (B // BLK,)
    full = lambda shape: pl.BlockSpec(shape, lambda i: (0,) * len(shape))
    return pl.pallas_call(
        _mlp_body,
        grid=grid,
        in_specs=[
            pl.BlockSpec((BLK, 256), lambda i: (i, 0)),
            pl.BlockSpec((BLK, 256), lambda i: (i, 0)),
            pl.BlockSpec((BLK, 1), lambda i: (i, 0)),
            pl.BlockSpec((BLK, 1), lambda i: (i, 0)),
            full((D, 128)),
            full((D, 128)),
            full((1, 128)),
            full((128, D)),
            full((1, D)),
            full((1, D)),
            full((1, 1)),
        ],
        out_specs=pl.BlockSpec((BLK, 1), lambda i: (i, 0)),
        out_shape=jax.ShapeDtypeStruct((B, 1), jnp.float32),
    )(su, si, hu, hi, w1u, w1i, b1r, w2t, b2r, w3r, b3r)


def kernel(user_ids, item_ids, user_table, item_table, W1, b1, W2, b2, W3, b3):
    uid = user_ids.astype(jnp.int32)
    iid = item_ids.astype(jnp.int32)
    u4 = lax.optimization_barrier(jnp.reshape(user_table, (NSLAB, 256)))
    i4 = jnp.reshape(item_table, (NSLAB, 256))
    su, si = _sc_gather(uid, iid, u4, i4)

    w1u = W1[:, :D].T
    w1i = W1[:, D:].T
    hu = (uid & 3).astype(jnp.float32).reshape(B, 1)
    hi = (iid & 3).astype(jnp.float32).reshape(B, 1)

    out = _mlp(su, si, hu, hi, w1u, w1i,
               b1.reshape(1, 128), W2.T, b2.reshape(1, D), W3,
               b3.reshape(1, 1))
    return out[:, 0]


# speedup vs baseline: 38.6244x; 1.0539x over previous
import functools

import jax
import jax.numpy as jnp
from jax import lax
from jax.experimental import pallas as pl
from jax.experimental.pallas import tpu as pltpu
from jax.experimental.pallas import tpu_sc as plsc

B = 16384
D = 64
NC = 2
NS = 16
NW = NC * NS
BPW = B // NW

_sc_mesh = plsc.VectorSubcoreMesh(core_axis_name="c", subcore_axis_name="s")


@functools.partial(
    pl.kernel,
    out_type=(
        jax.ShapeDtypeStruct((B, D), jnp.float32),
        jax.ShapeDtypeStruct((B, D), jnp.float32),
    ),
    mesh=_sc_mesh,
    compiler_params=pltpu.CompilerParams(use_tc_tiling_on_sc=False),
    scratch_types=[
        pltpu.VMEM((BPW,), jnp.int32),
        pltpu.VMEM((BPW,), jnp.int32),
        pltpu.VMEM((BPW, D), jnp.float32),
        pltpu.VMEM((BPW, D), jnp.float32),
        pltpu.SemaphoreType.DMA,
        pltpu.SemaphoreType.DMA,
    ],
)
def _sc_gather2(uid_hbm, iid_hbm, ut_hbm, it_hbm, ue_hbm, ie_hbm,
                uidx_v, iidx_v, urows_v, irows_v, sem_u, sem_i):
    wid = lax.axis_index("s") * NC + lax.axis_index("c")
    base = wid * BPW
    pltpu.sync_copy(uid_hbm.at[pl.ds(base, BPW)], uidx_v)
    pltpu.sync_copy(iid_hbm.at[pl.ds(base, BPW)], iidx_v)
    cu = pltpu.async_copy(ut_hbm.at[uidx_v], urows_v, sem_u)
    ci = pltpu.async_copy(it_hbm.at[iidx_v], irows_v, sem_i)
    cu.wait()
    ci.wait()
    pltpu.sync_copy(urows_v, ue_hbm.at[pl.ds(base, BPW)])
    pltpu.sync_copy(irows_v, ie_hbm.at[pl.ds(base, BPW)])


BLK = 2048


def _mlp_body(ue_ref, ie_ref, w1u_ref, w1i_ref, b1_ref, w2_ref, b2_ref,
              w3_ref, b3_ref, out_ref):
    x1 = (jnp.dot(ue_ref[...], w1u_ref[...], preferred_element_type=jnp.float32)
          + jnp.dot(ie_ref[...], w1i_ref[...], preferred_element_type=jnp.float32)
          + b1_ref[...])
    h1 = jnp.maximum(x1, 0.0)
    h2 = jnp.maximum(
        jnp.dot(h1, w2_ref[...], preferred_element_type=jnp.float32) + b2_ref[...],
        0.0)
    z = jnp.sum(h2 * w3_ref[...], axis=1, keepdims=True) + b3_ref[0, 0]
    out_ref[...] = 1.0 / (1.0 + jnp.exp(-z))


def _mlp(ue, ie, w1u, w1i, b1r, w2t, b2r, w3r, b3r):
    grid = (B // BLK,)
    full = lambda shape: pl.BlockSpec(shape, lambda i: (0, 0))
    return pl.pallas_call(
        _mlp_body,
        grid=grid,
        in_specs=[
            pl.BlockSpec((BLK, D), lambda i: (i, 0)),
            pl.BlockSpec((BLK, D), lambda i: (i, 0)),
            full((D, 128)),
            full((D, 128)),
            full((1, 128)),
            full((128, D)),
            full((1, D)),
            full((1, D)),
            full((1, 1)),
        ],
        out_specs=pl.BlockSpec((BLK, 1), lambda i: (i, 0)),
        out_shape=jax.ShapeDtypeStruct((B, 1), jnp.float32),
    )(ue, ie, w1u, w1i, b1r, w2t, b2r, w3r, b3r)


def kernel(user_ids, item_ids, user_table, item_table, W1, b1, W2, b2, W3, b3):
    uid = user_ids.astype(jnp.int32)
    iid = item_ids.astype(jnp.int32)
    ue, ie = _sc_gather2(uid, iid, user_table, item_table)
    w1u = W1[:, :D].T
    w1i = W1[:, D:].T
    w2t = W2.T
    out = _mlp(ue, ie, w1u, w1i, b1.reshape(1, 128), w2t, b2.reshape(1, D),
               W3, b3.reshape(1, 1))
    return out[:, 0]
